# baseline (speedup 1.0000x reference)
import functools

import jax
import jax.numpy as jnp
from jax import lax
from jax.experimental import pallas as pl
from jax.experimental.pallas import tpu as pltpu
from jax.experimental.pallas import tpu_sc as plsc

_DIM = 1024
_NE = 8
_NG = 4
_BLK = 2048
_NTOK = 32768

_NC = 2
_NS = 16
_NW = _NC * _NS
_TW = _NTOK // _NW
_L = 16


def _tc_scores_body(x_ref, wt_ref, st_ref):
    x = x_ref[...]
    wt = wt_ref[...]
    raw = jax.lax.dot_general(
        x, wt, (((1,), (0,)), ((), ())),
        preferred_element_type=jnp.float32)
    st_ref[...] = jax.nn.sigmoid(raw).T


def _tc_scores(x, wt):
    n_tok = x.shape[0]
    return pl.pallas_call(
        _tc_scores_body,
        grid=(n_tok // _BLK,),
        in_specs=[
            pl.BlockSpec((_BLK, _DIM), lambda i: (i, 0)),
            pl.BlockSpec((_DIM, _NE), lambda i: (0, 0)),
        ],
        out_specs=pl.BlockSpec((_NE, _BLK), lambda i: (0, i)),
        out_shape=jax.ShapeDtypeStruct((_NE, n_tok), jnp.float32),
    )(x, wt)


def _sc_route_body(tw, st_hbm, w0_hbm, w1_hbm, i0_hbm, i1_hbm,
                   s_v, w0_v, w1_v, i0_v, i1_v):
    wid = lax.axis_index("s") * _NC + lax.axis_index("c")
    base = wid * tw
    pltpu.sync_copy(st_hbm.at[:, pl.ds(base, tw)], s_v)

    neg = jnp.full((_L,), -jnp.inf, jnp.float32)
    one = jnp.full((_L,), 1, jnp.int32)
    zero = jnp.full((_L,), 0, jnp.int32)

    def step(t, carry):
        off = t * _L
        s = [s_v[e, pl.ds(off, _L)] for e in range(_NE)]
        g = [jnp.maximum(s[2 * j], s[2 * j + 1]) for j in range(_NG)]
        cnt = [zero] * _NG
        for j in range(_NG):
            for k in range(j + 1, _NG):
                jk = g[j] >= g[k]
                cnt[k] = cnt[k] + jnp.where(jk, one, zero)
                cnt[j] = cnt[j] + jnp.where(jk, zero, one)
        sel = [cnt[j] < 2 for j in range(_NG)]
        m = [jnp.where(sel[e // 2], s[e], neg) for e in range(_NE)]
        best, bidx = m[0], zero
        for e in range(1, _NE):
            c = m[e] > best
            best = jnp.where(c, m[e], best)
            bidx = jnp.where(c, jnp.full((_L,), e, jnp.int32), bidx)
        m2 = [jnp.where(bidx == e, neg, m[e]) for e in range(_NE)]
        best2, bidx2 = m2[0], zero
        for e in range(1, _NE):
            c = m2[e] > best2
            best2 = jnp.where(c, m2[e], best2)
            bidx2 = jnp.where(c, jnp.full((_L,), e, jnp.int32), bidx2)
        tot = best + best2
        w0_v[pl.ds(off, _L)] = best / tot
        w1_v[pl.ds(off, _L)] = best2 / tot
        i0_v[pl.ds(off, _L)] = bidx
        i1_v[pl.ds(off, _L)] = bidx2
        return carry

    lax.fori_loop(0, tw // _L, step, 0)

    pltpu.sync_copy(w0_v, w0_hbm.at[pl.ds(base, tw)])
    pltpu.sync_copy(w1_v, w1_hbm.at[pl.ds(base, tw)])
    pltpu.sync_copy(i0_v, i0_hbm.at[pl.ds(base, tw)])
    pltpu.sync_copy(i1_v, i1_hbm.at[pl.ds(base, tw)])


def _sc_route(st):
    n_tok = st.shape[1]
    tw = n_tok // _NW
    mesh = plsc.VectorSubcoreMesh(core_axis_name="c", subcore_axis_name="s")
    f = functools.partial(
        pl.kernel,
        out_type=[
            jax.ShapeDtypeStruct((n_tok,), jnp.float32),
            jax.ShapeDtypeStruct((n_tok,), jnp.float32),
            jax.ShapeDtypeStruct((n_tok,), jnp.int32),
            jax.ShapeDtypeStruct((n_tok,), jnp.int32),
        ],
        mesh=mesh,
        scratch_types=[
            pltpu.VMEM((_NE, tw), jnp.float32),
            pltpu.VMEM((tw,), jnp.float32),
            pltpu.VMEM((tw,), jnp.float32),
            pltpu.VMEM((tw,), jnp.int32),
            pltpu.VMEM((tw,), jnp.int32),
        ],
    )(functools.partial(_sc_route_body, tw))
    return f(st)


_NCHUNK = 2


@jax.jit
def kernel(x, weight):
    n_tok = x.shape[0]
    csz = n_tok // _NCHUNK
    wt = weight.T
    outs = []
    for c in range(_NCHUNK):
        st = _tc_scores(x[c * csz:(c + 1) * csz], wt)
        outs.append(_sc_route(st))
    w0, w1, i0, i1 = [jnp.concatenate([o[k] for o in outs]) for k in range(4)]
    weights = jnp.stack([w0, w1], axis=1)
    indices = jnp.stack([i0, i1], axis=1)
    return weights, indices


# speedup vs baseline: 1.2903x; 1.2903x over previous
import functools

import jax
import jax.numpy as jnp
from jax import lax
from jax.experimental import pallas as pl
from jax.experimental.pallas import tpu as pltpu
from jax.experimental.pallas import tpu_sc as plsc

_DIM = 1024
_NE = 8
_NG = 4
_BLK = 2048

_NC = 2
_NS = 16
_NW = _NC * _NS
_L = 16


def _tc_scores_body(x_ref, w_ref, st_ref):
    x = x_ref[...]
    w = w_ref[...]
    raw = jax.lax.dot_general(
        x, w, (((1,), (1,)), ((), ())),
        preferred_element_type=jnp.float32)
    st_ref[...] = jax.nn.sigmoid(raw).T


def _tc_scores(x, w):
    n_tok = x.shape[0]
    return pl.pallas_call(
        _tc_scores_body,
        grid=(n_tok // _BLK,),
        in_specs=[
            pl.BlockSpec((_BLK, _DIM), lambda i: (i, 0)),
            pl.BlockSpec((_NE, _DIM), lambda i: (0, 0)),
        ],
        out_specs=pl.BlockSpec((_NE, _BLK), lambda i: (0, i)),
        out_shape=jax.ShapeDtypeStruct((_NE, n_tok), jnp.float32),
    )(x, w)


def _sc_route_body(tw, st_hbm, w_hbm, i_hbm, s_v, w_v, i_v):
    wid = lax.axis_index("s") * _NC + lax.axis_index("c")
    base = wid * tw
    pltpu.sync_copy(st_hbm.at[:, pl.ds(base, tw)], s_v)

    neg = jnp.full((_L,), -jnp.inf, jnp.float32)
    one = jnp.full((_L,), 1, jnp.int32)
    zero = jnp.full((_L,), 0, jnp.int32)
    lane = lax.iota(jnp.int32, _L)
    half_lo = lax.shift_right_logical(lane, 1)
    half_hi = half_lo + 8
    evenl = (lane & 1) == 0
    def take1d(a, idx):
        return lax.gather(
            a, idx[:, None],
            dimension_numbers=lax.GatherDimensionNumbers(
                offset_dims=(), collapsed_slice_dims=(0,),
                start_index_map=(0,)),
            slice_sizes=(1,),
            mode=lax.GatherScatterMode.PROMISE_IN_BOUNDS)

    def interleave(a, b):
        lo = jnp.where(evenl, take1d(a, half_lo), take1d(b, half_lo))
        hi = jnp.where(evenl, take1d(a, half_hi), take1d(b, half_hi))
        return lo, hi

    def step(t, carry):
        off = t * _L
        s = [s_v[e, pl.ds(off, _L)] for e in range(_NE)]
        g = [jnp.maximum(s[2 * j], s[2 * j + 1]) for j in range(_NG)]
        cnt = [zero] * _NG
        for j in range(_NG):
            for k in range(j + 1, _NG):
                jk = g[j] >= g[k]
                cnt[k] = cnt[k] + jnp.where(jk, one, zero)
                cnt[j] = cnt[j] + jnp.where(jk, zero, one)
        sel = [cnt[j] < 2 for j in range(_NG)]
        m = [jnp.where(sel[e // 2], s[e], neg) for e in range(_NE)]
        best, bidx = m[0], zero
        for e in range(1, _NE):
            c = m[e] > best
            best = jnp.where(c, m[e], best)
            bidx = jnp.where(c, jnp.full((_L,), e, jnp.int32), bidx)
        m2 = [jnp.where(bidx == e, neg, m[e]) for e in range(_NE)]
        best2, bidx2 = m2[0], zero
        for e in range(1, _NE):
            c = m2[e] > best2
            best2 = jnp.where(c, m2[e], best2)
            bidx2 = jnp.where(c, jnp.full((_L,), e, jnp.int32), bidx2)
        tot = best + best2
        w_lo, w_hi = interleave(best / tot, best2 / tot)
        i_lo, i_hi = interleave(bidx, bidx2)
        w_v[pl.ds(2 * off, _L)] = w_lo
        w_v[pl.ds(2 * off + _L, _L)] = w_hi
        i_v[pl.ds(2 * off, _L)] = i_lo
        i_v[pl.ds(2 * off + _L, _L)] = i_hi
        return carry

    lax.fori_loop(0, tw // _L, step, 0)

    pltpu.sync_copy(w_v, w_hbm.at[pl.ds(2 * base, 2 * tw)])
    pltpu.sync_copy(i_v, i_hbm.at[pl.ds(2 * base, 2 * tw)])


def _sc_route(st):
    n_tok = st.shape[1]
    tw = n_tok // _NW
    mesh = plsc.VectorSubcoreMesh(core_axis_name="c", subcore_axis_name="s")
    f = functools.partial(
        pl.kernel,
        out_type=[
            jax.ShapeDtypeStruct((2 * n_tok,), jnp.float32),
            jax.ShapeDtypeStruct((2 * n_tok,), jnp.int32),
        ],
        mesh=mesh,
        scratch_types=[
            pltpu.VMEM((_NE, tw), jnp.float32),
            pltpu.VMEM((2 * tw,), jnp.float32),
            pltpu.VMEM((2 * tw,), jnp.int32),
        ],
    )(functools.partial(_sc_route_body, tw))
    return f(st)


@jax.jit
def kernel(x, weight):
    n_tok = x.shape[0]
    st = _tc_scores(x, weight)
    w_flat, i_flat = _sc_route(st)
    return w_flat.reshape(n_tok, 2), i_flat.reshape(n_tok, 2)


# speedup vs baseline: 2.2994x; 1.7820x over previous
import functools

import jax
import jax.numpy as jnp
from jax import lax
from jax.experimental import pallas as pl
from jax.experimental.pallas import tpu as pltpu
from jax.experimental.pallas import tpu_sc as plsc

_DIM = 1024
_NE = 8
_NG = 4
_BLK = 2048

_NC = 2
_NS = 16
_NW = _NC * _NS
_L = 16


def _tc_scores_body(x_ref, w_ref, st_ref):
    x = x_ref[...]
    w = w_ref[...]
    raw = jax.lax.dot_general(
        x, w, (((1,), (1,)), ((), ())),
        preferred_element_type=jnp.float32)
    st_ref[...] = jax.nn.sigmoid(raw).T


def _tc_scores(x, w):
    n_tok = x.shape[0]
    return pl.pallas_call(
        _tc_scores_body,
        grid=(n_tok // _BLK,),
        in_specs=[
            pl.BlockSpec((_BLK, _DIM), lambda i: (i, 0)),
            pl.BlockSpec((_NE, _DIM), lambda i: (0, 0)),
        ],
        out_specs=pl.BlockSpec((_NE, _BLK), lambda i: (0, i)),
        out_shape=jax.ShapeDtypeStruct((_NE, n_tok), jnp.float32),
    )(x, w)


def _sc_route_body(tw, st_hbm, w0_hbm, w1_hbm, i0_hbm, i1_hbm,
                   s_v, w0_v, w1_v, i0_v, i1_v):
    wid = lax.axis_index("s") * _NC + lax.axis_index("c")
    base = wid * tw
    pltpu.sync_copy(st_hbm.at[:, pl.ds(base, tw)], s_v)

    neg = jnp.full((_L,), -jnp.inf, jnp.float32)
    one = jnp.full((_L,), 1, jnp.int32)
    zero = jnp.full((_L,), 0, jnp.int32)
    def step(t, carry):
        off = t * _L
        s = [s_v[e, pl.ds(off, _L)] for e in range(_NE)]
        g = [jnp.maximum(s[2 * j], s[2 * j + 1]) for j in range(_NG)]
        cnt = [zero] * _NG
        for j in range(_NG):
            for k in range(j + 1, _NG):
                jk = g[j] >= g[k]
                cnt[k] = cnt[k] + jnp.where(jk, one, zero)
                cnt[j] = cnt[j] + jnp.where(jk, zero, one)
        sel = [cnt[j] < 2 for j in range(_NG)]
        m = [jnp.where(sel[e // 2], s[e], neg) for e in range(_NE)]
        best, bidx = m[0], zero
        for e in range(1, _NE):
            c = m[e] > best
            best = jnp.where(c, m[e], best)
            bidx = jnp.where(c, jnp.full((_L,), e, jnp.int32), bidx)
        m2 = [jnp.where(bidx == e, neg, m[e]) for e in range(_NE)]
        best2, bidx2 = m2[0], zero
        for e in range(1, _NE):
            c = m2[e] > best2
            best2 = jnp.where(c, m2[e], best2)
            bidx2 = jnp.where(c, jnp.full((_L,), e, jnp.int32), bidx2)
        tot = best + best2
        w0_v[pl.ds(off, _L)] = best / tot
        w1_v[pl.ds(off, _L)] = best2 / tot
        i0_v[pl.ds(off, _L)] = bidx
        i1_v[pl.ds(off, _L)] = bidx2
        return carry

    lax.fori_loop(0, tw // _L, step, 0)

    pltpu.sync_copy(w0_v, w0_hbm.at[pl.ds(base, tw)])
    pltpu.sync_copy(w1_v, w1_hbm.at[pl.ds(base, tw)])
    pltpu.sync_copy(i0_v, i0_hbm.at[pl.ds(base, tw)])
    pltpu.sync_copy(i1_v, i1_hbm.at[pl.ds(base, tw)])


def _sc_route(st):
    n_tok = st.shape[1]
    tw = n_tok // _NW
    mesh = plsc.VectorSubcoreMesh(core_axis_name="c", subcore_axis_name="s")
    f = functools.partial(
        pl.kernel,
        out_type=[
            jax.ShapeDtypeStruct((n_tok,), jnp.float32),
            jax.ShapeDtypeStruct((n_tok,), jnp.float32),
            jax.ShapeDtypeStruct((n_tok,), jnp.int32),
            jax.ShapeDtypeStruct((n_tok,), jnp.int32),
        ],
        mesh=mesh,
        scratch_types=[
            pltpu.VMEM((_NE, tw), jnp.float32),
            pltpu.VMEM((tw,), jnp.float32),
            pltpu.VMEM((tw,), jnp.float32),
            pltpu.VMEM((tw,), jnp.int32),
            pltpu.VMEM((tw,), jnp.int32),
        ],
    )(functools.partial(_sc_route_body, tw))
    return f(st)


@jax.jit
def kernel(x, weight):
    st = _tc_scores(x, weight)
    w0, w1, i0, i1 = _sc_route(st)
    return jnp.stack([w0, w1], axis=1), jnp.stack([i0, i1], axis=1)
